# baseline (speedup 1.0000x reference)
import functools

import jax
import jax.numpy as jnp
from jax import lax
from jax.experimental import pallas as pl
from jax.experimental.pallas import tpu as pltpu
from jax.experimental.pallas import tpu_sc as plsc

B = 16384
D = 32
L0, L1, L2 = 1000, 100000, 1000000
_NC = 2
_NS = 16
_NW = _NC * _NS
_V = 16
_NVEC = B // _V
_CH = 32768
_TAIL = L2 % 128
_L2A = L2 - _TAIL
_NCHUNK = -(-_L2A // _CH)
_Q = 4096
_NQ = B // _Q


def _gather_body(ids0, ids1, ids2, e0, e1, e2, o0, o1, o2, s_in, s_out):
    f = lax.axis_index("s") * _NC + lax.axis_index("c")

    def phase_emb2(idsb, dstb, tbl_a, tbl_b):
        pltpu.sync_copy(ids2, idsb)
        halves = (tbl_a, tbl_b)
        sizes = [min(_CH, _L2A - c * _CH) for c in range(_NCHUNK)]
        cps = [None, None]
        cps[0] = pltpu.async_copy(
            e2.at[f, pl.ds(0, sizes[0])], halves[0].at[pl.ds(0, sizes[0])],
            s_in)
        for c in range(_NCHUNK):
            if c + 1 < _NCHUNK:
                n = sizes[c + 1]
                cps[(c + 1) % 2] = pltpu.async_copy(
                    e2.at[f, pl.ds((c + 1) * _CH, n)],
                    halves[(c + 1) % 2].at[pl.ds(0, n)], s_in)
            cps[c % 2].wait()
            half = halves[c % 2]

            def scan(v, _, half=half, c=c):
                ids = idsb[pl.ds(v * _V, _V)]
                m = lax.shift_right_logical(ids, 15) == c
                loc = jnp.bitwise_and(ids, _CH - 1)
                g = plsc.load_gather(half, [loc], mask=m)
                pos = lax.iota(jnp.int32, _V) + v * _V
                plsc.store_scatter(dstb, [pos], g, mask=m)
                return 0

            lax.fori_loop(0, _NVEC, scan, 0, unroll=16)
        pltpu.sync_copy(dstb, o2.at[f])

    pl.run_scoped(
        phase_emb2,
        pltpu.VMEM((B,), jnp.int32),
        pltpu.VMEM((B,), jnp.float32),
        pltpu.VMEM((_CH,), jnp.float32),
        pltpu.VMEM((_CH,), jnp.float32),
    )

    def phase_emb01(row0, row1, idsb, dstb):
        pltpu.sync_copy(e0.at[f], row0)
        pltpu.sync_copy(e1.at[f], row1)
        for ids_hbm, row, out in ((ids0, row0, o0), (ids1, row1, o1)):
            for q in range(_NQ):
                pltpu.sync_copy(ids_hbm.at[pl.ds(q * _Q, _Q)], idsb)

                def lookup(v, _, row=row):
                    g = plsc.load_gather(row, [idsb[pl.ds(v * _V, _V)]])
                    dstb[pl.ds(v * _V, _V)] = g
                    return 0

                lax.fori_loop(0, _Q // _V, lookup, 0, unroll=8)
                pltpu.sync_copy(dstb, out.at[f, pl.ds(q * _Q, _Q)])

    pl.run_scoped(
        phase_emb01,
        pltpu.VMEM((L0,), jnp.float32),
        pltpu.VMEM((L1,), jnp.float32),
        pltpu.VMEM((_Q,), jnp.int32),
        pltpu.VMEM((_Q,), jnp.float32),
    )


_base_t = jax.ShapeDtypeStruct((D, B), jnp.float32)

_gather_call = functools.partial(
    pl.kernel,
    mesh=plsc.VectorSubcoreMesh(core_axis_name="c", subcore_axis_name="s"),
    compiler_params=pltpu.CompilerParams(needs_layout_passes=False),
    out_type=(_base_t, _base_t, _base_t),
    scratch_types=[
        pltpu.SemaphoreType.DMA,
        pltpu.SemaphoreType.DMA,
    ],
)(_gather_body)


def _proj_body(b0_ref, b1_ref, b2_ref, ids2_ref, tail_ref,
               w10_ref, w21_ref, w12_ref, bias_ref,
               o0_ref, o1_ref, o2_ref):
    b0 = b0_ref[...]
    b1 = b1_ref[...]
    blk = b2_ref.shape[1]
    ids2 = ids2_ref[0, :]
    rowid = lax.broadcasted_iota(jnp.int32, (_TAIL, blk), 0) + _L2A
    onehot = (rowid == ids2[None, :]).astype(jnp.float32)
    tail_b2 = jnp.dot(tail_ref[...], onehot,
                      preferred_element_type=jnp.float32)
    in_tail = (ids2 >= _L2A)[None, :]
    b2 = jnp.where(in_tail, tail_b2, b2_ref[...])
    o0_ref[...] = b0 + jnp.dot(w10_ref[...], b1,
                               preferred_element_type=jnp.float32) + bias_ref[:, 0:1]
    o1_ref[...] = b1 + jnp.dot(w21_ref[...], b2,
                               preferred_element_type=jnp.float32) + bias_ref[:, 1:2]
    o2_ref[...] = b2 + jnp.dot(w12_ref[...], b1,
                               preferred_element_type=jnp.float32) + bias_ref[:, 2:3]


_BLK = 2048
_outT_t = jax.ShapeDtypeStruct((D, B), jnp.float32)

_proj_call = pl.pallas_call(
    _proj_body,
    grid=(B // _BLK,),
    in_specs=[
        pl.BlockSpec((D, _BLK), lambda i: (0, i)),
        pl.BlockSpec((D, _BLK), lambda i: (0, i)),
        pl.BlockSpec((D, _BLK), lambda i: (0, i)),
        pl.BlockSpec((1, _BLK), lambda i: (0, i)),
        pl.BlockSpec((D, _TAIL), lambda i: (0, 0)),
        pl.BlockSpec((D, D), lambda i: (0, 0)),
        pl.BlockSpec((D, D), lambda i: (0, 0)),
        pl.BlockSpec((D, D), lambda i: (0, 0)),
        pl.BlockSpec((D, 8), lambda i: (0, 0)),
    ],
    out_specs=[
        pl.BlockSpec((D, _BLK), lambda i: (0, i)),
        pl.BlockSpec((D, _BLK), lambda i: (0, i)),
        pl.BlockSpec((D, _BLK), lambda i: (0, i)),
    ],
    out_shape=(_outT_t, _outT_t, _outT_t),
)


def kernel(level_ids_0, level_ids_1, level_ids_2, emb0, emb1, emb2,
           W01, b01, W10, b10, W12, b12, W21, b21):
    del W01, b01
    ids0 = level_ids_0.astype(jnp.int32)
    ids1 = level_ids_1.astype(jnp.int32)
    ids2 = level_ids_2.astype(jnp.int32)
    b0t, b1t, b2t = _gather_call(ids0, ids1, ids2, emb0.T, emb1.T, emb2.T)
    tail = emb2.T[:, _L2A:]
    bias = jnp.zeros((D, 8), jnp.float32)
    bias = bias.at[:, 0].set(b10).at[:, 1].set(b21).at[:, 2].set(b12)
    e0t, e1t, e2t = _proj_call(b0t, b1t, b2t, ids2.reshape(1, B), tail,
                               W10, W21, W12, bias)
    return (e0t.T, e1t.T, e2t.T)


# speedup vs baseline: 1.0340x; 1.0340x over previous
import functools

import jax
import jax.numpy as jnp
from jax import lax
from jax.experimental import pallas as pl
from jax.experimental.pallas import tpu as pltpu
from jax.experimental.pallas import tpu_sc as plsc

B = 16384
D = 32
L0, L1, L2 = 1000, 100000, 1000000
_NC = 2
_NS = 16
_NW = _NC * _NS
_V = 16
_NVEC = B // _V
_CH = 32768
_TAIL = L2 % 128
_L2A = L2 - _TAIL
_NCHUNK = -(-_L2A // _CH)
_Q = 4096
_NQ = B // _Q


def _gather_body(ids0, ids1, ids2, e0, e1, e2, o0, o1, o2, s_in, s_out):
    f = lax.axis_index("s") * _NC + lax.axis_index("c")

    def phase_emb2(idsb, dstb, tbl_a, tbl_b):
        pltpu.sync_copy(ids2, idsb)
        halves = (tbl_a, tbl_b)
        sizes = [min(_CH, _L2A - c * _CH) for c in range(_NCHUNK)]
        cps = [None, None]
        cps[0] = pltpu.async_copy(
            e2.at[f, pl.ds(0, sizes[0])], halves[0].at[pl.ds(0, sizes[0])],
            s_in)
        for c in range(_NCHUNK):
            if c + 1 < _NCHUNK:
                n = sizes[c + 1]
                cps[(c + 1) % 2] = pltpu.async_copy(
                    e2.at[f, pl.ds((c + 1) * _CH, n)],
                    halves[(c + 1) % 2].at[pl.ds(0, n)], s_in)
            cps[c % 2].wait()
            half = halves[c % 2]

            def scan(v, _, half=half, c=c):
                ids = idsb[pl.ds(v * _V, _V)]
                m = lax.shift_right_logical(ids, 15) == c
                loc = jnp.bitwise_and(ids, _CH - 1)
                g = plsc.load_gather(half, [loc], mask=m)
                pos = lax.iota(jnp.int32, _V) + v * _V
                plsc.store_scatter(dstb, [pos], g, mask=m)
                return 0

            lax.fori_loop(0, _NVEC, scan, 0, unroll=8)
        pltpu.sync_copy(dstb, o2.at[f])

    pl.run_scoped(
        phase_emb2,
        pltpu.VMEM((B,), jnp.int32),
        pltpu.VMEM((B,), jnp.float32),
        pltpu.VMEM((_CH,), jnp.float32),
        pltpu.VMEM((_CH,), jnp.float32),
    )

    def phase_emb01(row0, row1, idsb, dstb):
        pltpu.sync_copy(e0.at[f], row0)
        pltpu.sync_copy(e1.at[f], row1)
        for ids_hbm, row, out in ((ids0, row0, o0), (ids1, row1, o1)):
            for q in range(_NQ):
                pltpu.sync_copy(ids_hbm.at[pl.ds(q * _Q, _Q)], idsb)

                def lookup(v, _, row=row):
                    g = plsc.load_gather(row, [idsb[pl.ds(v * _V, _V)]])
                    dstb[pl.ds(v * _V, _V)] = g
                    return 0

                lax.fori_loop(0, _Q // _V, lookup, 0, unroll=8)
                pltpu.sync_copy(dstb, out.at[f, pl.ds(q * _Q, _Q)])

    pl.run_scoped(
        phase_emb01,
        pltpu.VMEM((L0,), jnp.float32),
        pltpu.VMEM((L1,), jnp.float32),
        pltpu.VMEM((_Q,), jnp.int32),
        pltpu.VMEM((_Q,), jnp.float32),
    )


_base_t = jax.ShapeDtypeStruct((D, B), jnp.float32)

_gather_call = functools.partial(
    pl.kernel,
    mesh=plsc.VectorSubcoreMesh(core_axis_name="c", subcore_axis_name="s"),
    compiler_params=pltpu.CompilerParams(needs_layout_passes=False),
    out_type=(_base_t, _base_t, _base_t),
    scratch_types=[
        pltpu.SemaphoreType.DMA,
        pltpu.SemaphoreType.DMA,
    ],
)(_gather_body)


def _proj_body(b0_ref, b1_ref, b2_ref, ids2_ref, tail_ref,
               w10_ref, w21_ref, w12_ref, bias_ref,
               o0_ref, o1_ref, o2_ref):
    b0 = b0_ref[...]
    b1 = b1_ref[...]
    blk = b2_ref.shape[1]
    ids2 = ids2_ref[0, :]
    rowid = lax.broadcasted_iota(jnp.int32, (_TAIL, blk), 0) + _L2A
    onehot = (rowid == ids2[None, :]).astype(jnp.float32)
    tail_b2 = jnp.dot(tail_ref[...], onehot,
                      preferred_element_type=jnp.float32)
    in_tail = (ids2 >= _L2A)[None, :]
    b2 = jnp.where(in_tail, tail_b2, b2_ref[...])
    o0_ref[...] = b0 + jnp.dot(w10_ref[...], b1,
                               preferred_element_type=jnp.float32) + bias_ref[:, 0:1]
    o1_ref[...] = b1 + jnp.dot(w21_ref[...], b2,
                               preferred_element_type=jnp.float32) + bias_ref[:, 1:2]
    o2_ref[...] = b2 + jnp.dot(w12_ref[...], b1,
                               preferred_element_type=jnp.float32) + bias_ref[:, 2:3]


_BLK = 4096
_outT_t = jax.ShapeDtypeStruct((D, B), jnp.float32)

_proj_call = pl.pallas_call(
    _proj_body,
    grid=(B // _BLK,),
    in_specs=[
        pl.BlockSpec((D, _BLK), lambda i: (0, i)),
        pl.BlockSpec((D, _BLK), lambda i: (0, i)),
        pl.BlockSpec((D, _BLK), lambda i: (0, i)),
        pl.BlockSpec((1, _BLK), lambda i: (0, i)),
        pl.BlockSpec((D, _TAIL), lambda i: (0, 0)),
        pl.BlockSpec((D, D), lambda i: (0, 0)),
        pl.BlockSpec((D, D), lambda i: (0, 0)),
        pl.BlockSpec((D, D), lambda i: (0, 0)),
        pl.BlockSpec((D, 8), lambda i: (0, 0)),
    ],
    out_specs=[
        pl.BlockSpec((D, _BLK), lambda i: (0, i)),
        pl.BlockSpec((D, _BLK), lambda i: (0, i)),
        pl.BlockSpec((D, _BLK), lambda i: (0, i)),
    ],
    out_shape=(_outT_t, _outT_t, _outT_t),
)


def kernel(level_ids_0, level_ids_1, level_ids_2, emb0, emb1, emb2,
           W01, b01, W10, b10, W12, b12, W21, b21):
    del W01, b01
    ids0 = level_ids_0.astype(jnp.int32)
    ids1 = level_ids_1.astype(jnp.int32)
    ids2 = level_ids_2.astype(jnp.int32)
    b0t, b1t, b2t = _gather_call(ids0, ids1, ids2, emb0.T, emb1.T, emb2.T)
    tail = emb2.T[:, _L2A:]
    bias = jnp.zeros((D, 8), jnp.float32)
    bias = bias.at[:, 0].set(b10).at[:, 1].set(b21).at[:, 2].set(b12)
    e0t, e1t, e2t = _proj_call(b0t, b1t, b2t, ids2.reshape(1, B), tail,
                               W10, W21, W12, bias)
    return (e0t.T, e1t.T, e2t.T)


# speedup vs baseline: 1.0448x; 1.0104x over previous
import functools

import jax
import jax.numpy as jnp
from jax import lax
from jax.experimental import pallas as pl
from jax.experimental.pallas import tpu as pltpu
from jax.experimental.pallas import tpu_sc as plsc

B = 16384
D = 32
L0, L1, L2 = 1000, 100000, 1000000
_NC = 2
_NS = 16
_NW = _NC * _NS
_V = 16
_NVEC = B // _V
_CH = 32768
_TAIL = L2 % 128
_L2A = L2 - _TAIL
_NCHUNK = -(-_L2A // _CH)
_Q = 4096
_NQ = B // _Q


def _gather_body(ids0, ids1, ids2, e0, e1, e2, o0, o1, o2, s_in, s_out):
    f = lax.axis_index("s") * _NC + lax.axis_index("c")

    def phase_emb2(idsb, dstb, tbl_a, tbl_b):
        pltpu.sync_copy(ids2, idsb)
        halves = (tbl_a, tbl_b)
        sizes = [min(_CH, _L2A - c * _CH) for c in range(_NCHUNK)]
        cps = [None, None]
        cps[0] = pltpu.async_copy(
            e2.at[f, pl.ds(0, sizes[0])], halves[0].at[pl.ds(0, sizes[0])],
            s_in)
        for c in range(_NCHUNK):
            if c + 1 < _NCHUNK:
                n = sizes[c + 1]
                cps[(c + 1) % 2] = pltpu.async_copy(
                    e2.at[f, pl.ds((c + 1) * _CH, n)],
                    halves[(c + 1) % 2].at[pl.ds(0, n)], s_in)
            cps[c % 2].wait()
            half = halves[c % 2]

            def scan(v, _, half=half, c=c):
                ids = idsb[pl.ds(v * _V, _V)]
                m = lax.shift_right_logical(ids, 15) == c
                loc = jnp.bitwise_and(ids, _CH - 1)
                g = plsc.load_gather(half, [loc], mask=m)
                pos = lax.iota(jnp.int32, _V) + v * _V
                plsc.store_scatter(dstb, [pos], g, mask=m)
                return 0

            lax.fori_loop(0, _NVEC, scan, 0, unroll=8)
        pltpu.sync_copy(dstb, o2.at[f])

    pl.run_scoped(
        phase_emb2,
        pltpu.VMEM((B,), jnp.int32),
        pltpu.VMEM((B,), jnp.float32),
        pltpu.VMEM((_CH,), jnp.float32),
        pltpu.VMEM((_CH,), jnp.float32),
    )

    def phase_emb01(row0, row1, idsb, dstb):
        pltpu.sync_copy(e0.at[f], row0)
        pltpu.sync_copy(e1.at[f], row1)
        for ids_hbm, row, out in ((ids0, row0, o0), (ids1, row1, o1)):
            for q in range(_NQ):
                pltpu.sync_copy(ids_hbm.at[pl.ds(q * _Q, _Q)], idsb)

                def lookup(v, _, row=row):
                    g = plsc.load_gather(row, [idsb[pl.ds(v * _V, _V)]])
                    dstb[pl.ds(v * _V, _V)] = g
                    return 0

                lax.fori_loop(0, _Q // _V, lookup, 0, unroll=8)
                pltpu.sync_copy(dstb, out.at[f, pl.ds(q * _Q, _Q)])

    pl.run_scoped(
        phase_emb01,
        pltpu.VMEM((L0,), jnp.float32),
        pltpu.VMEM((L1,), jnp.float32),
        pltpu.VMEM((_Q,), jnp.int32),
        pltpu.VMEM((_Q,), jnp.float32),
    )


_base_t = jax.ShapeDtypeStruct((D, B), jnp.float32)

_gather_call = functools.partial(
    pl.kernel,
    mesh=plsc.VectorSubcoreMesh(core_axis_name="c", subcore_axis_name="s"),
    compiler_params=pltpu.CompilerParams(needs_layout_passes=False),
    out_type=(_base_t, _base_t, _base_t),
    scratch_types=[
        pltpu.SemaphoreType.DMA,
        pltpu.SemaphoreType.DMA,
    ],
)(_gather_body)


def _proj_body(b0_ref, b1_ref, b2_ref, ids2_ref, tail_ref,
               w10_ref, w21_ref, w12_ref, bias_ref,
               o0_ref, o1_ref, o2_ref):
    b0 = b0_ref[...]
    b1 = b1_ref[...]
    blk = b2_ref.shape[1]
    ids2 = ids2_ref[0, :]
    rowid = lax.broadcasted_iota(jnp.int32, (_TAIL, blk), 0) + _L2A
    onehot = (rowid == ids2[None, :]).astype(jnp.float32)
    tail_b2 = jnp.dot(tail_ref[...], onehot,
                      preferred_element_type=jnp.float32)
    in_tail = (ids2 >= _L2A)[None, :]
    b2 = jnp.where(in_tail, tail_b2, b2_ref[...])
    o0_ref[...] = b0 + jnp.dot(w10_ref[...], b1,
                               preferred_element_type=jnp.float32) + bias_ref[:, 0:1]
    o1_ref[...] = b1 + jnp.dot(w21_ref[...], b2,
                               preferred_element_type=jnp.float32) + bias_ref[:, 1:2]
    o2_ref[...] = b2 + jnp.dot(w12_ref[...], b1,
                               preferred_element_type=jnp.float32) + bias_ref[:, 2:3]


_BLK = 8192
_outT_t = jax.ShapeDtypeStruct((D, B), jnp.float32)

_proj_call = pl.pallas_call(
    _proj_body,
    grid=(B // _BLK,),
    in_specs=[
        pl.BlockSpec((D, _BLK), lambda i: (0, i)),
        pl.BlockSpec((D, _BLK), lambda i: (0, i)),
        pl.BlockSpec((D, _BLK), lambda i: (0, i)),
        pl.BlockSpec((1, _BLK), lambda i: (0, i)),
        pl.BlockSpec((D, _TAIL), lambda i: (0, 0)),
        pl.BlockSpec((D, D), lambda i: (0, 0)),
        pl.BlockSpec((D, D), lambda i: (0, 0)),
        pl.BlockSpec((D, D), lambda i: (0, 0)),
        pl.BlockSpec((D, 8), lambda i: (0, 0)),
    ],
    out_specs=[
        pl.BlockSpec((D, _BLK), lambda i: (0, i)),
        pl.BlockSpec((D, _BLK), lambda i: (0, i)),
        pl.BlockSpec((D, _BLK), lambda i: (0, i)),
    ],
    out_shape=(_outT_t, _outT_t, _outT_t),
)


def kernel(level_ids_0, level_ids_1, level_ids_2, emb0, emb1, emb2,
           W01, b01, W10, b10, W12, b12, W21, b21):
    del W01, b01
    ids0 = level_ids_0.astype(jnp.int32)
    ids1 = level_ids_1.astype(jnp.int32)
    ids2 = level_ids_2.astype(jnp.int32)
    b0t, b1t, b2t = _gather_call(ids0, ids1, ids2, emb0.T, emb1.T, emb2.T)
    tail = emb2.T[:, _L2A:]
    bias = jnp.zeros((D, 8), jnp.float32)
    bias = bias.at[:, 0].set(b10).at[:, 1].set(b21).at[:, 2].set(b12)
    e0t, e1t, e2t = _proj_call(b0t, b1t, b2t, ids2.reshape(1, B), tail,
                               W10, W21, W12, bias)
    return (e0t.T, e1t.T, e2t.T)


# speedup vs baseline: 1.0786x; 1.0323x over previous
import functools

import jax
import jax.numpy as jnp
from jax import lax
from jax.experimental import pallas as pl
from jax.experimental.pallas import tpu as pltpu
from jax.experimental.pallas import tpu_sc as plsc

B = 16384
D = 32
L0, L1, L2 = 1000, 100000, 1000000
_NC = 2
_NS = 16
_NW = _NC * _NS
_V = 16
_NVEC = B // _V
_CH = 32768
_TAIL = L2 % 128
_L2A = L2 - _TAIL
_NCHUNK = -(-_L2A // _CH)
_Q = 8192
_NQ = B // _Q


def _gather_body(ids0, ids1, ids2, e0, e1, e2, o0, o1, o2, s_in, s_out):
    f = lax.axis_index("s") * _NC + lax.axis_index("c")

    def phase_emb2(idsb, dstb, tbl_a, tbl_b):
        pltpu.sync_copy(ids2, idsb)
        halves = (tbl_a, tbl_b)
        sizes = [min(_CH, _L2A - c * _CH) for c in range(_NCHUNK)]
        cps = [None, None]
        cps[0] = pltpu.async_copy(
            e2.at[f, pl.ds(0, sizes[0])], halves[0].at[pl.ds(0, sizes[0])],
            s_in)
        for c in range(_NCHUNK):
            if c + 1 < _NCHUNK:
                n = sizes[c + 1]
                cps[(c + 1) % 2] = pltpu.async_copy(
                    e2.at[f, pl.ds((c + 1) * _CH, n)],
                    halves[(c + 1) % 2].at[pl.ds(0, n)], s_in)
            cps[c % 2].wait()
            half = halves[c % 2]

            def scan(v, _, half=half, c=c):
                ids = idsb[pl.ds(v * _V, _V)]
                m = lax.shift_right_logical(ids, 15) == c
                loc = jnp.bitwise_and(ids, _CH - 1)
                g = plsc.load_gather(half, [loc], mask=m)
                pos = lax.iota(jnp.int32, _V) + v * _V
                plsc.store_scatter(dstb, [pos], g, mask=m)
                return 0

            lax.fori_loop(0, _NVEC, scan, 0, unroll=8)
        pltpu.sync_copy(dstb, o2.at[f])

    pl.run_scoped(
        phase_emb2,
        pltpu.VMEM((B,), jnp.int32),
        pltpu.VMEM((B,), jnp.float32),
        pltpu.VMEM((_CH,), jnp.float32),
        pltpu.VMEM((_CH,), jnp.float32),
    )

    def phase_emb01(row0, row1, idsb, dstb):
        pltpu.sync_copy(e0.at[f], row0)
        pltpu.sync_copy(e1.at[f], row1)
        for ids_hbm, row, out in ((ids0, row0, o0), (ids1, row1, o1)):
            for q in range(_NQ):
                pltpu.sync_copy(ids_hbm.at[pl.ds(q * _Q, _Q)], idsb)

                def lookup(v, _, row=row):
                    g = plsc.load_gather(row, [idsb[pl.ds(v * _V, _V)]])
                    dstb[pl.ds(v * _V, _V)] = g
                    return 0

                lax.fori_loop(0, _Q // _V, lookup, 0, unroll=8)
                pltpu.sync_copy(dstb, out.at[f, pl.ds(q * _Q, _Q)])

    pl.run_scoped(
        phase_emb01,
        pltpu.VMEM((L0,), jnp.float32),
        pltpu.VMEM((L1,), jnp.float32),
        pltpu.VMEM((_Q,), jnp.int32),
        pltpu.VMEM((_Q,), jnp.float32),
    )


_base_t = jax.ShapeDtypeStruct((D, B), jnp.float32)

_gather_call = functools.partial(
    pl.kernel,
    mesh=plsc.VectorSubcoreMesh(core_axis_name="c", subcore_axis_name="s"),
    compiler_params=pltpu.CompilerParams(needs_layout_passes=False),
    out_type=(_base_t, _base_t, _base_t),
    scratch_types=[
        pltpu.SemaphoreType.DMA,
        pltpu.SemaphoreType.DMA,
    ],
)(_gather_body)


def _proj_body(b0_ref, b1_ref, b2_ref, ids2_ref, tail_ref,
               w10_ref, w21_ref, w12_ref, bias_ref,
               o0_ref, o1_ref, o2_ref):
    b0 = b0_ref[...]
    b1 = b1_ref[...]
    blk = b2_ref.shape[1]
    ids2 = ids2_ref[0, :]
    rowid = lax.broadcasted_iota(jnp.int32, (_TAIL, blk), 0) + _L2A
    onehot = (rowid == ids2[None, :]).astype(jnp.float32)
    tail_b2 = jnp.dot(tail_ref[...], onehot,
                      preferred_element_type=jnp.float32)
    in_tail = (ids2 >= _L2A)[None, :]
    b2 = jnp.where(in_tail, tail_b2, b2_ref[...])
    o0_ref[...] = b0 + jnp.dot(w10_ref[...], b1,
                               preferred_element_type=jnp.float32) + bias_ref[:, 0:1]
    o1_ref[...] = b1 + jnp.dot(w21_ref[...], b2,
                               preferred_element_type=jnp.float32) + bias_ref[:, 1:2]
    o2_ref[...] = b2 + jnp.dot(w12_ref[...], b1,
                               preferred_element_type=jnp.float32) + bias_ref[:, 2:3]


_BLK = 8192
_outT_t = jax.ShapeDtypeStruct((D, B), jnp.float32)

_proj_call = pl.pallas_call(
    _proj_body,
    grid=(B // _BLK,),
    in_specs=[
        pl.BlockSpec((D, _BLK), lambda i: (0, i)),
        pl.BlockSpec((D, _BLK), lambda i: (0, i)),
        pl.BlockSpec((D, _BLK), lambda i: (0, i)),
        pl.BlockSpec((1, _BLK), lambda i: (0, i)),
        pl.BlockSpec((D, _TAIL), lambda i: (0, 0)),
        pl.BlockSpec((D, D), lambda i: (0, 0)),
        pl.BlockSpec((D, D), lambda i: (0, 0)),
        pl.BlockSpec((D, D), lambda i: (0, 0)),
        pl.BlockSpec((D, 8), lambda i: (0, 0)),
    ],
    out_specs=[
        pl.BlockSpec((D, _BLK), lambda i: (0, i)),
        pl.BlockSpec((D, _BLK), lambda i: (0, i)),
        pl.BlockSpec((D, _BLK), lambda i: (0, i)),
    ],
    out_shape=(_outT_t, _outT_t, _outT_t),
)


def kernel(level_ids_0, level_ids_1, level_ids_2, emb0, emb1, emb2,
           W01, b01, W10, b10, W12, b12, W21, b21):
    del W01, b01
    ids0 = level_ids_0.astype(jnp.int32)
    ids1 = level_ids_1.astype(jnp.int32)
    ids2 = level_ids_2.astype(jnp.int32)
    b0t, b1t, b2t = _gather_call(ids0, ids1, ids2, emb0.T, emb1.T, emb2.T)
    tail = emb2.T[:, _L2A:]
    bias = jnp.zeros((D, 8), jnp.float32)
    bias = bias.at[:, 0].set(b10).at[:, 1].set(b21).at[:, 2].set(b12)
    e0t, e1t, e2t = _proj_call(b0t, b1t, b2t, ids2.reshape(1, B), tail,
                               W10, W21, W12, bias)
    return (e0t.T, e1t.T, e2t.T)
